# baseline (speedup 1.0000x reference)
import functools
import math

import jax
import jax.numpy as jnp
from jax import lax
from jax.experimental import pallas as pl
from jax.experimental.pallas import tpu as pltpu
from jax.experimental.pallas import tpu_sc as plsc

D_MODEL = 64
SCALE = math.sqrt(D_MODEL)
LANES = 16
NC, NS = 2, 16
NW = NC * NS
W = 128


def _sc_embed(x2d, lut):
    n_win_total = x2d.shape[0]
    n_win = n_win_total // NW
    n_rows = n_win_total * W

    mesh = plsc.VectorSubcoreMesh(core_axis_name="c", subcore_axis_name="s")

    @functools.partial(
        pl.kernel,
        out_type=jax.ShapeDtypeStruct((n_rows, D_MODEL), jnp.float32),
        mesh=mesh,
        scratch_types=[
            pltpu.VMEM((n_win, W), jnp.int32),
            pltpu.VMEM((W, D_MODEL), jnp.float32),
            pltpu.SemaphoreType.DMA,
        ],
        compiler_params=pltpu.CompilerParams(use_tc_tiling_on_sc=False),
    )
    def k(x_hbm, lut_hbm, out_hbm, idx_v, rows_v, sem):
        wid = lax.axis_index("s") * NC + lax.axis_index("c")
        win0 = wid * n_win
        pltpu.sync_copy(x_hbm.at[pl.ds(win0, n_win)], idx_v)

        @pl.loop(0, n_win)
        def _(w):
            pltpu.async_copy(lut_hbm.at[idx_v.at[w]], rows_v, sem).wait()

            @pl.loop(0, W)
            def _(r):
                for j in range(D_MODEL // LANES):
                    sl = (r, pl.ds(j * LANES, LANES))
                    rows_v[sl] = rows_v[sl] * SCALE

            pltpu.sync_copy(rows_v, out_hbm.at[pl.ds((win0 + w) * W, W)])

    return k(x2d, lut)


def kernel(x, lut):
    x2d = x.reshape(-1, W).astype(jnp.int32)
    out = _sc_embed(x2d, lut)
    return out.reshape(x.shape + (D_MODEL,))


# speedup vs baseline: 1.1988x; 1.1988x over previous
import functools
import math

import jax
import jax.numpy as jnp
from jax import lax
from jax.experimental import pallas as pl
from jax.experimental.pallas import tpu as pltpu
from jax.experimental.pallas import tpu_sc as plsc

D_MODEL = 64
SCALE = math.sqrt(D_MODEL)
LANES = 16
NC, NS = 2, 16
NW = NC * NS
W = 128
CW = 4
NBUF = 3
CHUNK = CW * W


def _sc_embed(x2d, lut):
    n_win_total = x2d.shape[0]
    n_win = n_win_total // NW
    n_chunks = n_win // CW
    n_rows = n_win_total * W
    rows_per_w = n_win * W

    mesh = plsc.VectorSubcoreMesh(core_axis_name="c", subcore_axis_name="s")

    @functools.partial(
        pl.kernel,
        out_type=jax.ShapeDtypeStruct((n_rows, D_MODEL), jnp.float32),
        mesh=mesh,
        scratch_types=[
            pltpu.VMEM((n_win, W), jnp.int32),
            pltpu.VMEM((NBUF, CHUNK, D_MODEL), jnp.float32),
            pltpu.SemaphoreType.DMA((NBUF,)),
            pltpu.SemaphoreType.DMA((NBUF,)),
        ],
        compiler_params=pltpu.CompilerParams(use_tc_tiling_on_sc=False),
    )
    def k(x_hbm, lut_hbm, out_hbm, idx_v, rows_v, gsem, ssem):
        wid = lax.axis_index("s") * NC + lax.axis_index("c")
        win0 = wid * n_win
        row0 = wid * rows_per_w
        pltpu.sync_copy(x_hbm.at[pl.ds(win0, n_win)], idx_v)

        def fire_gathers(c, b):
            return [
                pltpu.async_copy(
                    lut_hbm.at[idx_v.at[c * CW + i]],
                    rows_v.at[b, pl.ds(i * W, W)],
                    gsem.at[b],
                )
                for i in range(CW)
            ]

        def fire_store(c, b):
            return pltpu.async_copy(
                rows_v.at[b],
                out_hbm.at[pl.ds(row0 + c * CHUNK, CHUNK)],
                ssem.at[b],
            )

        gh = {0: fire_gathers(0, 0), 1: fire_gathers(1, 1)}
        th = {}
        for c in range(n_chunks):
            b = c % NBUF
            for h in gh.pop(c):
                h.wait()
            if c + 2 < n_chunks:
                if c >= 1:
                    th.pop(c - 1).wait()
                gh[c + 2] = fire_gathers(c + 2, (c + 2) % NBUF)

            @pl.loop(0, CHUNK, step=8)
            def _(r0):
                for dr in range(8):
                    for j in range(D_MODEL // LANES):
                        sl = (b, r0 + dr, pl.ds(j * LANES, LANES))
                        rows_v[sl] = rows_v[sl] * SCALE

            th[c] = fire_store(c, b)
        for c in sorted(th):
            th.pop(c).wait()

    return k(x2d, lut)


def kernel(x, lut):
    x2d = x.reshape(-1, W).astype(jnp.int32)
    out = _sc_embed(x2d, lut)
    return out.reshape(x.shape + (D_MODEL,))
